# Initial kernel scaffold; baseline (speedup 1.0000x reference)
import functools

import jax
import jax.numpy as jnp
from jax import lax
from jax.experimental import pallas as pl
from jax.experimental.pallas import tpu as pltpu
from jax.experimental.pallas import tpu_sc as plsc

N_NODES = 10000
N_EDGES = 320000
IN_CH = 128
HID = 16

NC = 2
NS = 16
NW = NC * NS
CHUNK = 128
EDGES_PAD = 327680
CHUNKS_PER_TILE = EDGES_PAD // (NW * CHUNK)
PAD_NODES = 10016
STRIPE = PAD_NODES // NS

_mesh = plsc.VectorSubcoreMesh(core_axis_name="c", subcore_axis_name="s")


def _zero_rows(ref, n):
    def body(i, _):
        ref[i] = jnp.zeros((HID,), jnp.float32)
        return 0
    lax.fori_loop(0, n, body, 0)


@functools.partial(
    pl.kernel,
    out_type=jax.ShapeDtypeStruct((NC, PAD_NODES, HID), jnp.float32),
    mesh=_mesh,
    scratch_types=[
        pltpu.VMEM((CHUNKS_PER_TILE, CHUNK), jnp.int32),
        pltpu.VMEM((CHUNK, HID), jnp.float32),
        pltpu.VMEM((STRIPE, HID), jnp.float32),
        pltpu.VMEM_SHARED((PAD_NODES, HID), jnp.float32),
    ],
)
def _deg_kernel(dst_hbm, out_hbm, dst_v, ones_v, zeros_v, acc_sh):
    cid = lax.axis_index("c")
    sid = lax.axis_index("s")
    wid = sid * NC + cid

    _zero_rows(zeros_v, STRIPE)

    def ones_body(i, _):
        ones_v[i] = jnp.ones((HID,), jnp.float32)
        return 0
    lax.fori_loop(0, CHUNK, ones_body, 0)

    pltpu.sync_copy(zeros_v, acc_sh.at[pl.ds(sid * STRIPE, STRIPE)])
    pltpu.sync_copy(dst_hbm.at[wid], dst_v)
    plsc.subcore_barrier()

    def body(j, _):
        pltpu.sync_copy(ones_v, acc_sh.at[dst_v.at[j]], add=True)
        return 0
    lax.fori_loop(0, CHUNKS_PER_TILE, body, 0)

    plsc.subcore_barrier()
    pltpu.sync_copy(acc_sh.at[pl.ds(sid * STRIPE, STRIPE)],
                    out_hbm.at[cid, pl.ds(sid * STRIPE, STRIPE)])


@functools.partial(
    pl.kernel,
    out_type=jax.ShapeDtypeStruct((NC, PAD_NODES, HID), jnp.float32),
    mesh=_mesh,
    scratch_types=[
        pltpu.VMEM((CHUNKS_PER_TILE, CHUNK), jnp.int32),
        pltpu.VMEM((CHUNKS_PER_TILE, CHUNK), jnp.int32),
        pltpu.VMEM((CHUNK, HID), jnp.float32),
        pltpu.VMEM((STRIPE, HID), jnp.float32),
        pltpu.VMEM_SHARED((PAD_NODES, HID), jnp.float32),
        pltpu.SemaphoreType.DMA,
    ],
)
def _scatter_kernel(y_hbm, src_hbm, dst_hbm, out_hbm,
                    src_v, dst_v, rows_v, zeros_v, acc_sh, sem):
    cid = lax.axis_index("c")
    sid = lax.axis_index("s")
    wid = sid * NC + cid

    _zero_rows(zeros_v, STRIPE)
    pltpu.sync_copy(zeros_v, acc_sh.at[pl.ds(sid * STRIPE, STRIPE)])
    pltpu.sync_copy(src_hbm.at[wid], src_v)
    pltpu.sync_copy(dst_hbm.at[wid], dst_v)
    plsc.subcore_barrier()

    def body(j, _):
        pltpu.async_copy(y_hbm.at[src_v.at[j]], rows_v, sem).wait()
        pltpu.sync_copy(rows_v, acc_sh.at[dst_v.at[j]], add=True)
        return 0
    lax.fori_loop(0, CHUNKS_PER_TILE, body, 0)

    plsc.subcore_barrier()
    pltpu.sync_copy(acc_sh.at[pl.ds(sid * STRIPE, STRIPE)],
                    out_hbm.at[cid, pl.ds(sid * STRIPE, STRIPE)])


def _tca_body(x_ref, w1_ref, degp_ref, y_ref, dinv_ref):
    xw = jnp.dot(x_ref[:], w1_ref[:], preferred_element_type=jnp.float32)
    deg = degp_ref[0, :N_NODES, :] + degp_ref[1, :N_NODES, :] + 1.0
    dinv = lax.rsqrt(deg)
    dinv_ref[:] = dinv
    y_ref[:] = xw * dinv


def _tcb_body(acc_ref, y_ref, dinv_ref, b1_ref, w2_ref, b2_ref, out_ref):
    s = acc_ref[0, :N_NODES, :] + acc_ref[1, :N_NODES, :] + y_ref[:]
    h = jnp.maximum(dinv_ref[:] * s + b1_ref[:], 0.0)
    out_ref[:] = (jnp.dot(h, w2_ref[:], preferred_element_type=jnp.float32)
                  + b2_ref[0, 0])


def kernel(x, edge_index, W1, b1, W2, b2):
    src = edge_index[0].astype(jnp.int32)
    dst = edge_index[1].astype(jnp.int32)
    pad = EDGES_PAD - N_EDGES
    src_p = jnp.concatenate([src, jnp.zeros((pad,), jnp.int32)])
    dst_p = jnp.concatenate([dst, jnp.full((pad,), N_NODES, jnp.int32)])
    src3 = src_p.reshape(NW, CHUNKS_PER_TILE, CHUNK)
    dst3 = dst_p.reshape(NW, CHUNKS_PER_TILE, CHUNK)

    degp = _deg_kernel(dst3)

    y, dinv = pl.pallas_call(
        _tca_body,
        out_shape=(
            jax.ShapeDtypeStruct((N_NODES, HID), jnp.float32),
            jax.ShapeDtypeStruct((N_NODES, HID), jnp.float32),
        ),
    )(x, W1, degp)

    acc = _scatter_kernel(y, src3, dst3)

    out = pl.pallas_call(
        _tcb_body,
        out_shape=jax.ShapeDtypeStruct((N_NODES, 1), jnp.float32),
    )(acc, y, dinv, b1.reshape(1, HID), W2, b2.reshape(1, 1))
    return out[:, 0]


# speedup vs baseline: 37.2417x; 37.2417x over previous
import functools

import jax
import jax.numpy as jnp
from jax import lax
from jax.experimental import pallas as pl
from jax.experimental.pallas import tpu as pltpu
from jax.experimental.pallas import tpu_sc as plsc

N_NODES = 10000
N_EDGES = 320000
IN_CH = 128
HID = 16

NC = 2
NS = 16
NW = NC * NS
CHUNK = 128
EDGES_PAD = 327680
CHUNKS_PER_TILE = EDGES_PAD // (NW * CHUNK)
PAD_NODES = 10112
STRIPE = PAD_NODES // NS

_mesh = plsc.VectorSubcoreMesh(core_axis_name="c", subcore_axis_name="s")


def _zero_rows(ref, n):
    def body(i, _):
        ref[i] = jnp.zeros((HID,), jnp.float32)
        return 0
    lax.fori_loop(0, n, body, 0)


@functools.partial(
    pl.kernel,
    out_type=jax.ShapeDtypeStruct((NC, PAD_NODES, HID), jnp.float32),
    mesh=_mesh,
    scratch_types=[
        pltpu.VMEM((CHUNKS_PER_TILE, CHUNK), jnp.int32),
        pltpu.VMEM((CHUNK, HID), jnp.float32),
        pltpu.VMEM((STRIPE, HID), jnp.float32),
        pltpu.VMEM_SHARED((PAD_NODES, HID), jnp.float32),
    ],
    compiler_params=pltpu.CompilerParams(use_tc_tiling_on_sc=False),
)
def _deg_kernel(dst_hbm, out_hbm, dst_v, ones_v, zeros_v, acc_sh):
    cid = lax.axis_index("c")
    sid = lax.axis_index("s")
    wid = sid * NC + cid

    _zero_rows(zeros_v, STRIPE)

    def ones_body(i, _):
        ones_v[i] = jnp.ones((HID,), jnp.float32)
        return 0
    lax.fori_loop(0, CHUNK, ones_body, 0)

    pltpu.sync_copy(zeros_v, acc_sh.at[pl.ds(sid * STRIPE, STRIPE)])
    pltpu.sync_copy(dst_hbm.at[wid], dst_v)
    plsc.subcore_barrier()

    def body(j, _):
        pltpu.sync_copy(ones_v, acc_sh.at[dst_v.at[j]], add=True)
        return 0
    lax.fori_loop(0, CHUNKS_PER_TILE, body, 0)

    plsc.subcore_barrier()
    pltpu.sync_copy(acc_sh.at[pl.ds(sid * STRIPE, STRIPE)],
                    out_hbm.at[cid, pl.ds(sid * STRIPE, STRIPE)])


@functools.partial(
    pl.kernel,
    out_type=jax.ShapeDtypeStruct((NC, PAD_NODES, HID), jnp.float32),
    mesh=_mesh,
    scratch_types=[
        pltpu.VMEM((CHUNKS_PER_TILE, CHUNK), jnp.int32),
        pltpu.VMEM((CHUNKS_PER_TILE, CHUNK), jnp.int32),
        pltpu.VMEM((CHUNK, HID), jnp.float32),
        pltpu.VMEM((STRIPE, HID), jnp.float32),
        pltpu.VMEM_SHARED((PAD_NODES, HID), jnp.float32),
        pltpu.SemaphoreType.DMA,
    ],
    compiler_params=pltpu.CompilerParams(use_tc_tiling_on_sc=False),
)
def _scatter_kernel(y_hbm, src_hbm, dst_hbm, out_hbm,
                    src_v, dst_v, rows_v, zeros_v, acc_sh, sem):
    cid = lax.axis_index("c")
    sid = lax.axis_index("s")
    wid = sid * NC + cid

    _zero_rows(zeros_v, STRIPE)
    pltpu.sync_copy(zeros_v, acc_sh.at[pl.ds(sid * STRIPE, STRIPE)])
    pltpu.sync_copy(src_hbm.at[wid], src_v)
    pltpu.sync_copy(dst_hbm.at[wid], dst_v)
    plsc.subcore_barrier()

    def body(j, _):
        pltpu.async_copy(y_hbm.at[src_v.at[j]], rows_v, sem).wait()
        pltpu.sync_copy(rows_v, acc_sh.at[dst_v.at[j]], add=True)
        return 0
    lax.fori_loop(0, CHUNKS_PER_TILE, body, 0)

    plsc.subcore_barrier()
    pltpu.sync_copy(acc_sh.at[pl.ds(sid * STRIPE, STRIPE)],
                    out_hbm.at[cid, pl.ds(sid * STRIPE, STRIPE)])


def _tca_body(x_ref, w1_ref, degp_ref, y_ref, dinv_ref):
    xw = jnp.dot(x_ref[:], w1_ref[:], preferred_element_type=jnp.float32)
    deg = degp_ref[0, :N_NODES, :] + degp_ref[1, :N_NODES, :] + 1.0
    dinv = lax.rsqrt(deg)
    dinv_ref[:] = dinv
    y_ref[:] = xw * dinv


def _tcb_body(acc_ref, y_ref, dinv_ref, b1_ref, w2_ref, b2_ref, out_ref):
    s = acc_ref[0, :N_NODES, :] + acc_ref[1, :N_NODES, :] + y_ref[:]
    h = jnp.maximum(dinv_ref[:] * s + b1_ref[:], 0.0)
    out_ref[:] = (jnp.dot(h, w2_ref[:], preferred_element_type=jnp.float32)
                  + b2_ref[0, 0])


def kernel(x, edge_index, W1, b1, W2, b2):
    src = edge_index[0].astype(jnp.int32)
    dst = edge_index[1].astype(jnp.int32)
    pad = EDGES_PAD - N_EDGES
    src_p = jnp.concatenate([src, jnp.zeros((pad,), jnp.int32)])
    dst_p = jnp.concatenate([dst, jnp.full((pad,), N_NODES, jnp.int32)])
    src3 = src_p.reshape(NW, CHUNKS_PER_TILE, CHUNK)
    dst3 = dst_p.reshape(NW, CHUNKS_PER_TILE, CHUNK)

    degp = _deg_kernel(dst3)

    y, dinv = pl.pallas_call(
        _tca_body,
        out_shape=(
            jax.ShapeDtypeStruct((N_NODES, HID), jnp.float32),
            jax.ShapeDtypeStruct((N_NODES, HID), jnp.float32),
        ),
    )(x, W1, degp)

    acc = _scatter_kernel(y, src3, dst3)

    out = pl.pallas_call(
        _tcb_body,
        out_shape=jax.ShapeDtypeStruct((N_NODES, 1), jnp.float32),
    )(acc, y, dinv, b1.reshape(1, HID), W2, b2.reshape(1, 1))
    return out[:, 0]


# speedup vs baseline: 45.7385x; 1.2282x over previous
import functools

import jax
import jax.numpy as jnp
from jax import lax
from jax.experimental import pallas as pl
from jax.experimental.pallas import tpu as pltpu
from jax.experimental.pallas import tpu_sc as plsc

N_NODES = 10000
N_EDGES = 320000
IN_CH = 128
HID = 16

NC = 2
NS = 16
NW = NC * NS
CHUNK = 128
EDGES_PAD = 327680
CHUNKS_PER_TILE = EDGES_PAD // (NW * CHUNK)
PAD_NODES = 10112
STRIPE = PAD_NODES // NS
NBUF = 8
GROUPS = CHUNKS_PER_TILE // NBUF

_mesh = plsc.VectorSubcoreMesh(core_axis_name="c", subcore_axis_name="s")


def _zero_rows(ref, n):
    def body(i, _):
        ref[i] = jnp.zeros((HID,), jnp.float32)
        return 0
    lax.fori_loop(0, n, body, 0)


@functools.partial(
    pl.kernel,
    out_type=jax.ShapeDtypeStruct((NC, PAD_NODES, HID), jnp.float32),
    mesh=_mesh,
    scratch_types=[
        pltpu.VMEM((CHUNKS_PER_TILE, CHUNK), jnp.int32),
        pltpu.VMEM((CHUNK, HID), jnp.float32),
        pltpu.VMEM((STRIPE, HID), jnp.float32),
        pltpu.VMEM_SHARED((PAD_NODES, HID), jnp.float32),
        pltpu.SemaphoreType.DMA,
    ],
    compiler_params=pltpu.CompilerParams(use_tc_tiling_on_sc=False),
)
def _deg_kernel(dst_hbm, out_hbm, dst_v, ones_v, zeros_v, acc_sh, sem):
    cid = lax.axis_index("c")
    sid = lax.axis_index("s")
    wid = sid * NC + cid

    _zero_rows(zeros_v, STRIPE)

    def ones_body(i, _):
        ones_v[i] = jnp.ones((HID,), jnp.float32)
        return 0
    lax.fori_loop(0, CHUNK, ones_body, 0)

    pltpu.sync_copy(zeros_v, acc_sh.at[pl.ds(sid * STRIPE, STRIPE)])
    pltpu.sync_copy(dst_hbm.at[wid], dst_v)
    plsc.subcore_barrier()

    def fire(j, _):
        pltpu.async_copy(ones_v, acc_sh.at[dst_v.at[j]], sem, add=True)
        return 0
    lax.fori_loop(0, CHUNKS_PER_TILE, fire, 0)

    def drain(j, _):
        pltpu.make_async_copy(ones_v, acc_sh.at[dst_v.at[j]], sem).wait()
        return 0
    lax.fori_loop(0, CHUNKS_PER_TILE, drain, 0)

    plsc.subcore_barrier()
    pltpu.sync_copy(acc_sh.at[pl.ds(sid * STRIPE, STRIPE)],
                    out_hbm.at[cid, pl.ds(sid * STRIPE, STRIPE)])


@functools.partial(
    pl.kernel,
    out_type=jax.ShapeDtypeStruct((NC, PAD_NODES, HID), jnp.float32),
    mesh=_mesh,
    scratch_types=[
        pltpu.VMEM((CHUNKS_PER_TILE, CHUNK), jnp.int32),
        pltpu.VMEM((CHUNKS_PER_TILE, CHUNK), jnp.int32),
        pltpu.VMEM((2, NBUF, CHUNK, HID), jnp.float32),
        pltpu.VMEM((STRIPE, HID), jnp.float32),
        pltpu.VMEM_SHARED((PAD_NODES, HID), jnp.float32),
        pltpu.SemaphoreType.DMA,
        pltpu.SemaphoreType.DMA,
        pltpu.SemaphoreType.DMA,
        pltpu.SemaphoreType.DMA,
    ],
    compiler_params=pltpu.CompilerParams(use_tc_tiling_on_sc=False),
)
def _scatter_kernel(y_hbm, src_hbm, dst_hbm, out_hbm,
                    src_v, dst_v, rows_v, zeros_v, acc_sh,
                    gsem0, gsem1, ssem0, ssem1):
    cid = lax.axis_index("c")
    sid = lax.axis_index("s")
    wid = sid * NC + cid

    _zero_rows(zeros_v, STRIPE)
    pltpu.sync_copy(zeros_v, acc_sh.at[pl.ds(sid * STRIPE, STRIPE)])
    pltpu.sync_copy(src_hbm.at[wid], src_v)
    pltpu.sync_copy(dst_hbm.at[wid], dst_v)
    plsc.subcore_barrier()

    def fire_gathers(g, half, sem):
        base = g * NBUF
        for b in range(NBUF):
            pltpu.async_copy(y_hbm.at[src_v.at[base + b]], rows_v.at[half, b], sem)

    def drain_gathers(g, half, sem):
        base = g * NBUF
        for b in range(NBUF):
            pltpu.make_async_copy(y_hbm.at[src_v.at[base + b]],
                                  rows_v.at[half, b], sem).wait()

    def fire_scatters(g, half, sem):
        base = g * NBUF
        for b in range(NBUF):
            pltpu.async_copy(rows_v.at[half, b], acc_sh.at[dst_v.at[base + b]],
                             sem, add=True)

    def drain_scatters(g, half, sem):
        base = g * NBUF
        for b in range(NBUF):
            pltpu.make_async_copy(rows_v.at[half, b],
                                  acc_sh.at[dst_v.at[base + b]], sem).wait()

    fire_gathers(0, 0, gsem0)

    def body(gg, _):
        g0 = 2 * gg
        g1 = g0 + 1
        fire_gathers(g1, 1, gsem1)
        drain_gathers(g0, 0, gsem0)
        fire_scatters(g0, 0, ssem0)
        drain_scatters(g0, 0, ssem0)

        @pl.when(g0 + 2 < GROUPS)
        def _():
            fire_gathers(g0 + 2, 0, gsem0)

        drain_gathers(g1, 1, gsem1)
        fire_scatters(g1, 1, ssem1)
        drain_scatters(g1, 1, ssem1)
        return 0
    lax.fori_loop(0, GROUPS // 2, body, 0)

    plsc.subcore_barrier()
    pltpu.sync_copy(acc_sh.at[pl.ds(sid * STRIPE, STRIPE)],
                    out_hbm.at[cid, pl.ds(sid * STRIPE, STRIPE)])


def _tca_body(x_ref, w1_ref, degp_ref, y_ref, dinv_ref):
    xw = jnp.dot(x_ref[:], w1_ref[:], preferred_element_type=jnp.float32)
    deg = degp_ref[0, :N_NODES, :] + degp_ref[1, :N_NODES, :] + 1.0
    dinv = lax.rsqrt(deg)
    dinv_ref[:] = dinv
    y_ref[:] = xw * dinv


def _tcb_body(acc_ref, y_ref, dinv_ref, b1_ref, w2_ref, b2_ref, out_ref):
    s = acc_ref[0, :N_NODES, :] + acc_ref[1, :N_NODES, :] + y_ref[:]
    h = jnp.maximum(dinv_ref[:] * s + b1_ref[:], 0.0)
    out_ref[:] = (jnp.dot(h, w2_ref[:], preferred_element_type=jnp.float32)
                  + b2_ref[0, 0])


def kernel(x, edge_index, W1, b1, W2, b2):
    src = edge_index[0].astype(jnp.int32)
    dst = edge_index[1].astype(jnp.int32)
    pad = EDGES_PAD - N_EDGES
    src_p = jnp.concatenate([src, jnp.zeros((pad,), jnp.int32)])
    dst_p = jnp.concatenate([dst, jnp.full((pad,), N_NODES, jnp.int32)])
    src3 = src_p.reshape(NW, CHUNKS_PER_TILE, CHUNK)
    dst3 = dst_p.reshape(NW, CHUNKS_PER_TILE, CHUNK)

    degp = _deg_kernel(dst3)

    y, dinv = pl.pallas_call(
        _tca_body,
        out_shape=(
            jax.ShapeDtypeStruct((N_NODES, HID), jnp.float32),
            jax.ShapeDtypeStruct((N_NODES, HID), jnp.float32),
        ),
    )(x, W1, degp)

    acc = _scatter_kernel(y, src3, dst3)

    out = pl.pallas_call(
        _tcb_body,
        out_shape=jax.ShapeDtypeStruct((N_NODES, 1), jnp.float32),
    )(acc, y, dinv, b1.reshape(1, HID), W2, b2.reshape(1, 1))
    return out[:, 0]


# speedup vs baseline: 59.6627x; 1.3044x over previous
import functools

import jax
import jax.numpy as jnp
from jax import lax
from jax.experimental import pallas as pl
from jax.experimental.pallas import tpu as pltpu
from jax.experimental.pallas import tpu_sc as plsc

N_NODES = 10000
N_EDGES = 320000
IN_CH = 128
HID = 16

NC = 2
NS = 16
NW = NC * NS
CHUNK = 128
EDGES_PAD = 327680
CHUNKS_PER_TILE = EDGES_PAD // (NW * CHUNK)
PAD_NODES = 10112
STRIPE = PAD_NODES // NS
NBUF = 8
GROUPS = CHUNKS_PER_TILE // NBUF

_mesh = plsc.VectorSubcoreMesh(core_axis_name="c", subcore_axis_name="s")


def _zero_rows(ref, n):
    def body(i, _):
        ref[i] = jnp.zeros((HID,), jnp.float32)
        return 0
    lax.fori_loop(0, n, body, 0)


@functools.partial(
    pl.kernel,
    out_type=jax.ShapeDtypeStruct((NC, PAD_NODES, HID), jnp.float32),
    mesh=_mesh,
    scratch_types=[
        pltpu.VMEM((CHUNKS_PER_TILE, CHUNK), jnp.int32),
        pltpu.VMEM((CHUNK, HID), jnp.float32),
        pltpu.VMEM((STRIPE, HID), jnp.float32),
        pltpu.VMEM_SHARED((PAD_NODES, HID), jnp.float32),
        pltpu.SemaphoreType.DMA,
    ],
    compiler_params=pltpu.CompilerParams(use_tc_tiling_on_sc=False),
)
def _deg_kernel(dst_hbm, out_hbm, dst_v, ones_v, zeros_v, acc_sh, sem):
    cid = lax.axis_index("c")
    sid = lax.axis_index("s")
    wid = sid * NC + cid

    _zero_rows(zeros_v, STRIPE)

    def ones_body(i, _):
        ones_v[i] = jnp.ones((HID,), jnp.float32)
        return 0
    lax.fori_loop(0, CHUNK, ones_body, 0)

    pltpu.sync_copy(zeros_v, acc_sh.at[pl.ds(sid * STRIPE, STRIPE)])
    pltpu.sync_copy(dst_hbm.at[wid], dst_v)
    plsc.subcore_barrier()

    def fire(j, _):
        pltpu.async_copy(ones_v, acc_sh.at[dst_v.at[j]], sem, add=True)
        return 0
    lax.fori_loop(0, CHUNKS_PER_TILE, fire, 0)

    def drain(j, _):
        pltpu.make_async_copy(ones_v, acc_sh.at[dst_v.at[j]], sem).wait()
        return 0
    lax.fori_loop(0, CHUNKS_PER_TILE, drain, 0)

    plsc.subcore_barrier()
    pltpu.sync_copy(acc_sh.at[pl.ds(sid * STRIPE, STRIPE)],
                    out_hbm.at[cid, pl.ds(sid * STRIPE, STRIPE)])


@functools.partial(
    pl.kernel,
    out_type=jax.ShapeDtypeStruct((NC, PAD_NODES, HID), jnp.float32),
    mesh=_mesh,
    scratch_types=[
        pltpu.VMEM((CHUNKS_PER_TILE, CHUNK), jnp.int32),
        pltpu.VMEM((CHUNKS_PER_TILE, CHUNK), jnp.int32),
        pltpu.VMEM((2, NBUF, CHUNK, HID), jnp.float32),
        pltpu.VMEM((STRIPE, HID), jnp.float32),
        pltpu.VMEM_SHARED((PAD_NODES, HID), jnp.float32),
        pltpu.VMEM_SHARED((N_NODES, HID), jnp.float32),
        pltpu.SemaphoreType.DMA,
        pltpu.SemaphoreType.DMA,
        pltpu.SemaphoreType.DMA,
        pltpu.SemaphoreType.DMA,
    ],
    compiler_params=pltpu.CompilerParams(use_tc_tiling_on_sc=False),
)
def _scatter_kernel(y_hbm, src_hbm, dst_hbm, out_hbm,
                    src_v, dst_v, rows_v, zeros_v, acc_sh, y_sh,
                    gsem0, gsem1, ssem0, ssem1):
    cid = lax.axis_index("c")
    sid = lax.axis_index("s")
    wid = sid * NC + cid
    ystripe = N_NODES // NS

    _zero_rows(zeros_v, STRIPE)
    pltpu.sync_copy(zeros_v, acc_sh.at[pl.ds(sid * STRIPE, STRIPE)])
    pltpu.sync_copy(y_hbm.at[pl.ds(sid * ystripe, ystripe)],
                    y_sh.at[pl.ds(sid * ystripe, ystripe)])
    pltpu.sync_copy(src_hbm.at[wid], src_v)
    pltpu.sync_copy(dst_hbm.at[wid], dst_v)
    plsc.subcore_barrier()

    def fire_gathers(g, half, sem):
        base = g * NBUF
        for b in range(NBUF):
            pltpu.async_copy(y_sh.at[src_v.at[base + b]], rows_v.at[half, b], sem)

    def drain_gathers(g, half, sem):
        base = g * NBUF
        for b in range(NBUF):
            pltpu.make_async_copy(y_sh.at[src_v.at[base + b]],
                                  rows_v.at[half, b], sem).wait()

    def fire_scatters(g, half, sem):
        base = g * NBUF
        for b in range(NBUF):
            pltpu.async_copy(rows_v.at[half, b], acc_sh.at[dst_v.at[base + b]],
                             sem, add=True)

    def drain_scatters(g, half, sem):
        base = g * NBUF
        for b in range(NBUF):
            pltpu.make_async_copy(rows_v.at[half, b],
                                  acc_sh.at[dst_v.at[base + b]], sem).wait()

    fire_gathers(0, 0, gsem0)

    def body(gg, _):
        g0 = 2 * gg
        g1 = g0 + 1
        fire_gathers(g1, 1, gsem1)
        drain_gathers(g0, 0, gsem0)
        fire_scatters(g0, 0, ssem0)
        drain_scatters(g0, 0, ssem0)

        @pl.when(g0 + 2 < GROUPS)
        def _():
            fire_gathers(g0 + 2, 0, gsem0)

        drain_gathers(g1, 1, gsem1)
        fire_scatters(g1, 1, ssem1)
        drain_scatters(g1, 1, ssem1)
        return 0
    lax.fori_loop(0, GROUPS // 2, body, 0)

    plsc.subcore_barrier()
    pltpu.sync_copy(acc_sh.at[pl.ds(sid * STRIPE, STRIPE)],
                    out_hbm.at[cid, pl.ds(sid * STRIPE, STRIPE)])


def _tca_body(x_ref, w1_ref, degp_ref, y_ref, dinv_ref):
    xw = jnp.dot(x_ref[:], w1_ref[:], preferred_element_type=jnp.float32)
    deg = degp_ref[0, :N_NODES, :] + degp_ref[1, :N_NODES, :] + 1.0
    dinv = lax.rsqrt(deg)
    dinv_ref[:] = dinv
    y_ref[:] = xw * dinv


def _tcb_body(acc_ref, y_ref, dinv_ref, b1_ref, w2_ref, b2_ref, out_ref):
    s = acc_ref[0, :N_NODES, :] + acc_ref[1, :N_NODES, :] + y_ref[:]
    h = jnp.maximum(dinv_ref[:] * s + b1_ref[:], 0.0)
    out_ref[:] = (jnp.dot(h, w2_ref[:], preferred_element_type=jnp.float32)
                  + b2_ref[0, 0])


def kernel(x, edge_index, W1, b1, W2, b2):
    src = edge_index[0].astype(jnp.int32)
    dst = edge_index[1].astype(jnp.int32)
    pad = EDGES_PAD - N_EDGES
    src_p = jnp.concatenate([src, jnp.zeros((pad,), jnp.int32)])
    dst_p = jnp.concatenate([dst, jnp.full((pad,), N_NODES, jnp.int32)])
    src3 = src_p.reshape(NW, CHUNKS_PER_TILE, CHUNK)
    dst3 = dst_p.reshape(NW, CHUNKS_PER_TILE, CHUNK)

    degp = _deg_kernel(dst3)

    y, dinv = pl.pallas_call(
        _tca_body,
        out_shape=(
            jax.ShapeDtypeStruct((N_NODES, HID), jnp.float32),
            jax.ShapeDtypeStruct((N_NODES, HID), jnp.float32),
        ),
    )(x, W1, degp)

    acc = _scatter_kernel(y, src3, dst3)

    out = pl.pallas_call(
        _tcb_body,
        out_shape=jax.ShapeDtypeStruct((N_NODES, 1), jnp.float32),
    )(acc, y, dinv, b1.reshape(1, HID), W2, b2.reshape(1, 1))
    return out[:, 0]


# speedup vs baseline: 73.1131x; 1.2254x over previous
import functools

import jax
import jax.numpy as jnp
from jax import lax
from jax.experimental import pallas as pl
from jax.experimental.pallas import tpu as pltpu
from jax.experimental.pallas import tpu_sc as plsc

N_NODES = 10000
N_EDGES = 320000
IN_CH = 128
HID = 16

NC = 2
NS = 16
NW = NC * NS
EPT = N_EDGES // NW
CHUNK = 128
NFULL = EPT // CHUNK
REM = EPT - NFULL * CHUNK
PAD_NODES = 10112
STRIPE = PAD_NODES // NS
NBUF = 13
GROUPS = NFULL // NBUF

_mesh = plsc.VectorSubcoreMesh(core_axis_name="c", subcore_axis_name="s")


def _zero_rows(ref, n):
    def body(i, _):
        ref[i] = jnp.zeros((HID,), jnp.float32)
        return 0
    lax.fori_loop(0, n, body, 0)


@functools.partial(
    pl.kernel,
    out_type=jax.ShapeDtypeStruct((NC, PAD_NODES, HID), jnp.float32),
    mesh=_mesh,
    scratch_types=[
        pltpu.VMEM((EPT,), jnp.int32),
        pltpu.VMEM((CHUNK, HID), jnp.float32),
        pltpu.VMEM((STRIPE, HID), jnp.float32),
        pltpu.VMEM_SHARED((PAD_NODES, HID), jnp.float32),
        pltpu.SemaphoreType.DMA,
    ],
    compiler_params=pltpu.CompilerParams(use_tc_tiling_on_sc=False),
)
def _deg_kernel(edge_hbm, out_hbm, dst_v, ones_v, zeros_v, acc_sh, sem):
    cid = lax.axis_index("c")
    sid = lax.axis_index("s")
    wid = sid * NC + cid
    base = wid * EPT

    _zero_rows(zeros_v, STRIPE)

    def ones_body(i, _):
        ones_v[i] = jnp.ones((HID,), jnp.float32)
        return 0
    lax.fori_loop(0, CHUNK, ones_body, 0)

    pltpu.sync_copy(zeros_v, acc_sh.at[pl.ds(sid * STRIPE, STRIPE)])
    pltpu.sync_copy(edge_hbm.at[1, pl.ds(base, EPT)], dst_v)
    plsc.subcore_barrier()

    def fire(j, _):
        pltpu.async_copy(ones_v, acc_sh.at[dst_v.at[pl.ds(j * CHUNK, CHUNK)]],
                         sem, add=True)
        return 0
    lax.fori_loop(0, NFULL, fire, 0)
    pltpu.async_copy(ones_v.at[pl.ds(0, REM)],
                     acc_sh.at[dst_v.at[pl.ds(NFULL * CHUNK, REM)]], sem, add=True)

    def drain(j, _):
        pltpu.make_async_copy(ones_v, acc_sh.at[dst_v.at[pl.ds(0, CHUNK)]],
                              sem).wait()
        return 0
    lax.fori_loop(0, NFULL, drain, 0)
    pltpu.make_async_copy(ones_v.at[pl.ds(0, REM)],
                          acc_sh.at[dst_v.at[pl.ds(0, REM)]], sem).wait()

    plsc.subcore_barrier()
    pltpu.sync_copy(acc_sh.at[pl.ds(sid * STRIPE, STRIPE)],
                    out_hbm.at[cid, pl.ds(sid * STRIPE, STRIPE)])


@functools.partial(
    pl.kernel,
    out_type=jax.ShapeDtypeStruct((NC, PAD_NODES, HID), jnp.float32),
    mesh=_mesh,
    scratch_types=[
        pltpu.VMEM((EPT,), jnp.int32),
        pltpu.VMEM((EPT,), jnp.int32),
        pltpu.VMEM((2, NBUF, CHUNK, HID), jnp.float32),
        pltpu.VMEM((REM, HID), jnp.float32),
        pltpu.VMEM((STRIPE, HID), jnp.float32),
        pltpu.VMEM_SHARED((PAD_NODES, HID), jnp.float32),
        pltpu.VMEM_SHARED((N_NODES, HID), jnp.float32),
        pltpu.SemaphoreType.DMA,
        pltpu.SemaphoreType.DMA,
        pltpu.SemaphoreType.DMA,
        pltpu.SemaphoreType.DMA,
    ],
    compiler_params=pltpu.CompilerParams(use_tc_tiling_on_sc=False),
)
def _scatter_kernel(y_hbm, edge_hbm, out_hbm,
                    src_v, dst_v, rows_v, rem_v, zeros_v, acc_sh, y_sh,
                    gsem0, gsem1, ssem0, ssem1):
    cid = lax.axis_index("c")
    sid = lax.axis_index("s")
    wid = sid * NC + cid
    base = wid * EPT
    ystripe = N_NODES // NS

    _zero_rows(zeros_v, STRIPE)
    pltpu.sync_copy(zeros_v, acc_sh.at[pl.ds(sid * STRIPE, STRIPE)])
    pltpu.sync_copy(y_hbm.at[pl.ds(sid * ystripe, ystripe)],
                    y_sh.at[pl.ds(sid * ystripe, ystripe)])
    pltpu.sync_copy(edge_hbm.at[0, pl.ds(base, EPT)], src_v)
    pltpu.sync_copy(edge_hbm.at[1, pl.ds(base, EPT)], dst_v)
    plsc.subcore_barrier()

    def sidx(j):
        return src_v.at[pl.ds(j * CHUNK, CHUNK)]

    def didx(j):
        return dst_v.at[pl.ds(j * CHUNK, CHUNK)]

    def fire_gathers(g, half, sem):
        base_c = g * NBUF
        for b in range(NBUF):
            pltpu.async_copy(y_sh.at[sidx(base_c + b)], rows_v.at[half, b], sem)

    def drain_gathers(g, half, sem):
        base_c = g * NBUF
        for b in range(NBUF):
            pltpu.make_async_copy(y_sh.at[sidx(base_c + b)],
                                  rows_v.at[half, b], sem).wait()

    def fire_scatters(g, half, sem):
        base_c = g * NBUF
        for b in range(NBUF):
            pltpu.async_copy(rows_v.at[half, b], acc_sh.at[didx(base_c + b)],
                             sem, add=True)

    def drain_scatters(g, half, sem):
        base_c = g * NBUF
        for b in range(NBUF):
            pltpu.make_async_copy(rows_v.at[half, b],
                                  acc_sh.at[didx(base_c + b)], sem).wait()

    fire_gathers(0, 0, gsem0)

    def body(gg, _):
        g0 = 2 * gg
        g1 = g0 + 1
        fire_gathers(g1, 1, gsem1)
        drain_gathers(g0, 0, gsem0)
        fire_scatters(g0, 0, ssem0)
        drain_scatters(g0, 0, ssem0)

        @pl.when(g0 + 2 < GROUPS)
        def _():
            fire_gathers(g0 + 2, 0, gsem0)

        drain_gathers(g1, 1, gsem1)
        fire_scatters(g1, 1, ssem1)
        drain_scatters(g1, 1, ssem1)
        return 0
    lax.fori_loop(0, GROUPS // 2, body, 0)

    rs = src_v.at[pl.ds(NFULL * CHUNK, REM)]
    rd = dst_v.at[pl.ds(NFULL * CHUNK, REM)]
    pltpu.async_copy(y_sh.at[rs], rem_v, gsem0).wait()
    pltpu.async_copy(rem_v, acc_sh.at[rd], ssem0, add=True).wait()

    plsc.subcore_barrier()
    pltpu.sync_copy(acc_sh.at[pl.ds(sid * STRIPE, STRIPE)],
                    out_hbm.at[cid, pl.ds(sid * STRIPE, STRIPE)])


def _tca_body(x_ref, w1_ref, degp_ref, y_ref, dinv_ref):
    xw = jnp.dot(x_ref[:], w1_ref[:], preferred_element_type=jnp.float32)
    deg = degp_ref[0, :N_NODES, :] + degp_ref[1, :N_NODES, :] + 1.0
    dinv = lax.rsqrt(deg)
    dinv_ref[:] = dinv
    y_ref[:] = xw * dinv


def _tcb_body(acc_ref, y_ref, dinv_ref, b1_ref, w2_ref, b2_ref, out_ref):
    s = acc_ref[0, :N_NODES, :] + acc_ref[1, :N_NODES, :] + y_ref[:]
    h = jnp.maximum(dinv_ref[:] * s + b1_ref[:], 0.0)
    out_ref[:] = (jnp.dot(h, w2_ref[:], preferred_element_type=jnp.float32)
                  + b2_ref[0, 0])


def kernel(x, edge_index, W1, b1, W2, b2):
    e32 = edge_index.astype(jnp.int32)

    degp = _deg_kernel(e32)

    y, dinv = pl.pallas_call(
        _tca_body,
        out_shape=(
            jax.ShapeDtypeStruct((N_NODES, HID), jnp.float32),
            jax.ShapeDtypeStruct((N_NODES, HID), jnp.float32),
        ),
    )(x, W1, degp)

    acc = _scatter_kernel(y, e32)

    out = pl.pallas_call(
        _tcb_body,
        out_shape=jax.ShapeDtypeStruct((N_NODES, 1), jnp.float32),
    )(acc, y, dinv, b1.reshape(1, HID), W2, b2.reshape(1, 1))
    return out[:, 0]


# speedup vs baseline: 89.5418x; 1.2247x over previous
import functools

import jax
import jax.numpy as jnp
from jax import lax
from jax.experimental import pallas as pl
from jax.experimental.pallas import tpu as pltpu
from jax.experimental.pallas import tpu_sc as plsc

N_NODES = 10000
N_EDGES = 320000
IN_CH = 128
HID = 16

NC = 2
NS = 16
NW = NC * NS
EPT = N_EDGES // NW
CHUNK = 128
NFULL = EPT // CHUNK
REM = EPT - NFULL * CHUNK
PAD_NODES = 10112
STRIPE = PAD_NODES // NS
NBUF = 13
GROUPS = NFULL // NBUF

_mesh = plsc.VectorSubcoreMesh(core_axis_name="c", subcore_axis_name="s")


def _zero_rows(ref, n):
    def body(i, _):
        ref[i] = jnp.zeros((HID,), jnp.float32)
        return 0
    lax.fori_loop(0, n, body, 0)


@functools.partial(
    pl.kernel,
    out_type=jax.ShapeDtypeStruct((NC, PAD_NODES, HID), jnp.float32),
    mesh=_mesh,
    scratch_types=[
        pltpu.VMEM((EPT,), jnp.int32),
        pltpu.VMEM((CHUNK, HID), jnp.float32),
        pltpu.VMEM((STRIPE, HID), jnp.float32),
        pltpu.VMEM_SHARED((PAD_NODES, HID), jnp.float32),
        pltpu.SemaphoreType.DMA,
    ],
    compiler_params=pltpu.CompilerParams(use_tc_tiling_on_sc=False),
)
def _deg_kernel(edge_hbm, out_hbm, dst_v, ones_v, zeros_v, acc_sh, sem):
    cid = lax.axis_index("c")
    sid = lax.axis_index("s")
    wid = sid * NC + cid
    base = wid * EPT

    _zero_rows(zeros_v, STRIPE)

    def ones_body(i, _):
        ones_v[i] = jnp.ones((HID,), jnp.float32)
        return 0
    lax.fori_loop(0, CHUNK, ones_body, 0)

    pltpu.sync_copy(zeros_v, acc_sh.at[pl.ds(sid * STRIPE, STRIPE)])
    pltpu.sync_copy(edge_hbm.at[1, pl.ds(base, EPT)], dst_v)
    plsc.subcore_barrier()

    def fire(j, _):
        pltpu.async_copy(ones_v, acc_sh.at[dst_v.at[pl.ds(j * CHUNK, CHUNK)]],
                         sem, add=True)
        return 0
    lax.fori_loop(0, NFULL, fire, 0)
    pltpu.async_copy(ones_v.at[pl.ds(0, REM)],
                     acc_sh.at[dst_v.at[pl.ds(NFULL * CHUNK, REM)]], sem, add=True)

    def drain(j, _):
        pltpu.make_async_copy(ones_v, acc_sh.at[dst_v.at[pl.ds(0, CHUNK)]],
                              sem).wait()
        return 0
    lax.fori_loop(0, NFULL, drain, 0)
    pltpu.make_async_copy(ones_v.at[pl.ds(0, REM)],
                          acc_sh.at[dst_v.at[pl.ds(0, REM)]], sem).wait()

    plsc.subcore_barrier()
    pltpu.sync_copy(acc_sh.at[pl.ds(sid * STRIPE, STRIPE)],
                    out_hbm.at[cid, pl.ds(sid * STRIPE, STRIPE)])


@functools.partial(
    pl.kernel,
    out_type=jax.ShapeDtypeStruct((NC, PAD_NODES, HID), jnp.float32),
    mesh=_mesh,
    scratch_types=[
        pltpu.VMEM((EPT,), jnp.int32),
        pltpu.VMEM((EPT,), jnp.int32),
        pltpu.VMEM((2, NBUF, CHUNK, HID), jnp.float32),
        pltpu.VMEM((REM, HID), jnp.float32),
        pltpu.VMEM((STRIPE, HID), jnp.float32),
        pltpu.VMEM_SHARED((PAD_NODES, HID), jnp.float32),
        pltpu.VMEM_SHARED((PAD_NODES, HID), jnp.float32),
        pltpu.SemaphoreType.DMA,
        pltpu.SemaphoreType.DMA,
        pltpu.SemaphoreType.DMA,
        pltpu.SemaphoreType.DMA,
    ],
    compiler_params=pltpu.CompilerParams(use_tc_tiling_on_sc=False),
)
def _scatter_kernel(y_hbm, edge_hbm, out_hbm,
                    src_v, dst_v, rows_v, rem_v, zeros_v, acc_sh, y_sh,
                    gsem0, gsem1, ssem0, ssem1):
    cid = lax.axis_index("c")
    sid = lax.axis_index("s")
    wid = sid * NC + cid
    base = wid * EPT
    ystripe = PAD_NODES // NS

    _zero_rows(zeros_v, STRIPE)
    pltpu.sync_copy(zeros_v, acc_sh.at[pl.ds(sid * STRIPE, STRIPE)])
    pltpu.sync_copy(y_hbm.at[pl.ds(sid * ystripe, ystripe)],
                    y_sh.at[pl.ds(sid * ystripe, ystripe)])
    pltpu.sync_copy(edge_hbm.at[0, pl.ds(base, EPT)], src_v)
    pltpu.sync_copy(edge_hbm.at[1, pl.ds(base, EPT)], dst_v)
    plsc.subcore_barrier()

    def sidx(j):
        return src_v.at[pl.ds(j * CHUNK, CHUNK)]

    def didx(j):
        return dst_v.at[pl.ds(j * CHUNK, CHUNK)]

    def fire_gathers(g, half, sem):
        base_c = g * NBUF
        for b in range(NBUF):
            pltpu.async_copy(y_sh.at[sidx(base_c + b)], rows_v.at[half, b], sem)

    def drain_gathers(g, half, sem):
        base_c = g * NBUF
        for b in range(NBUF):
            pltpu.make_async_copy(y_sh.at[sidx(base_c + b)],
                                  rows_v.at[half, b], sem).wait()

    def fire_scatters(g, half, sem):
        base_c = g * NBUF
        for b in range(NBUF):
            pltpu.async_copy(rows_v.at[half, b], acc_sh.at[didx(base_c + b)],
                             sem, add=True)

    def drain_scatters(g, half, sem):
        base_c = g * NBUF
        for b in range(NBUF):
            pltpu.make_async_copy(rows_v.at[half, b],
                                  acc_sh.at[didx(base_c + b)], sem).wait()

    fire_gathers(0, 0, gsem0)

    def body(gg, _):
        g0 = 2 * gg
        g1 = g0 + 1
        fire_gathers(g1, 1, gsem1)
        drain_gathers(g0, 0, gsem0)
        fire_scatters(g0, 0, ssem0)
        drain_scatters(g0, 0, ssem0)

        @pl.when(g0 + 2 < GROUPS)
        def _():
            fire_gathers(g0 + 2, 0, gsem0)

        drain_gathers(g1, 1, gsem1)
        fire_scatters(g1, 1, ssem1)
        drain_scatters(g1, 1, ssem1)
        return 0
    lax.fori_loop(0, GROUPS // 2, body, 0)

    rs = src_v.at[pl.ds(NFULL * CHUNK, REM)]
    rd = dst_v.at[pl.ds(NFULL * CHUNK, REM)]
    pltpu.async_copy(y_sh.at[rs], rem_v, gsem0).wait()
    pltpu.async_copy(rem_v, acc_sh.at[rd], ssem0, add=True).wait()

    plsc.subcore_barrier()
    pltpu.sync_copy(acc_sh.at[pl.ds(sid * STRIPE, STRIPE)],
                    out_hbm.at[cid, pl.ds(sid * STRIPE, STRIPE)])


NFLAT = PAD_NODES * HID // 128
NOUT = PAD_NODES // 128


def _tca_body(x_ref, w1_ref, degp_ref, y_ref):
    xw = jnp.dot(x_ref[:], w1_ref[:], preferred_element_type=jnp.float32)
    deg = degp_ref[0, :N_NODES, :] + degp_ref[1, :N_NODES, :] + 1.0
    y_ref[:N_NODES] = xw * lax.rsqrt(deg)
    y_ref[N_NODES:] = jnp.zeros((PAD_NODES - N_NODES, HID), jnp.float32)


def _tcb_body(accf_ref, yf_ref, degf_ref, b1f_ref, w2bd_ref, b2_ref, out_ref):
    dinvf = lax.rsqrt(degf_ref[0] + degf_ref[1] + 1.0)
    s = accf_ref[0] + accf_ref[1] + yf_ref[:]
    h = jnp.maximum(dinvf * s + b1f_ref[:], 0.0)
    ovals = jnp.dot(h, w2bd_ref[:], preferred_element_type=jnp.float32)
    out_ref[:] = ovals + b2_ref[0, 0]


def kernel(x, edge_index, W1, b1, W2, b2):
    e32 = edge_index.astype(jnp.int32)

    degp = _deg_kernel(e32)

    y = pl.pallas_call(
        _tca_body,
        out_shape=jax.ShapeDtypeStruct((PAD_NODES, HID), jnp.float32),
    )(x, W1, degp)
    ylin = y.reshape(NFLAT, 128)

    acc = _scatter_kernel(ylin.reshape(PAD_NODES, HID), e32)
    accf = acc.reshape(NC, NFLAT, 128)
    degf = degp.reshape(NC, NFLAT, 128)

    w2bd = jnp.kron(jnp.eye(8, dtype=jnp.float32), W2)
    b1f = jnp.tile(b1, 8).reshape(1, 128)

    outp = pl.pallas_call(
        _tcb_body,
        out_shape=jax.ShapeDtypeStruct((NFLAT, 8), jnp.float32),
    )(accf, ylin, degf, b1f, w2bd, b2.reshape(1, 1))
    return outp.reshape(PAD_NODES)[:N_NODES]


# speedup vs baseline: 113.2616x; 1.2649x over previous
import functools

import jax
import jax.numpy as jnp
from jax import lax
from jax.experimental import pallas as pl
from jax.experimental.pallas import tpu as pltpu
from jax.experimental.pallas import tpu_sc as plsc

N_NODES = 10000
N_EDGES = 320000
IN_CH = 128
HID = 16

NC = 2
NS = 16
NW = NC * NS
EPT = N_EDGES // NW
CHUNK = 128
NFULL = EPT // CHUNK
REM = EPT - NFULL * CHUNK
PAD_NODES = 10112
STRIPE = PAD_NODES // NS
NBUF = 13
GROUPS = NFULL // NBUF

_mesh = plsc.VectorSubcoreMesh(core_axis_name="c", subcore_axis_name="s")


def _zero_rows(ref, n):
    def body(i, _):
        ref[i] = jnp.zeros((HID,), jnp.float32)
        return 0
    lax.fori_loop(0, n, body, 0)


@functools.partial(
    pl.kernel,
    out_type=jax.ShapeDtypeStruct((NC, PAD_NODES, HID), jnp.float32),
    mesh=_mesh,
    scratch_types=[
        pltpu.VMEM((EPT,), jnp.int32),
        pltpu.VMEM((CHUNK, HID), jnp.float32),
        pltpu.VMEM((STRIPE, HID), jnp.float32),
        pltpu.VMEM_SHARED((PAD_NODES, HID), jnp.float32),
        pltpu.SemaphoreType.DMA,
    ],
    compiler_params=pltpu.CompilerParams(use_tc_tiling_on_sc=False),
)
def _deg_kernel(edge_hbm, out_hbm, dst_v, ones_v, zeros_v, acc_sh, sem):
    cid = lax.axis_index("c")
    sid = lax.axis_index("s")
    wid = sid * NC + cid
    base = wid * EPT

    _zero_rows(zeros_v, STRIPE)

    def ones_body(i, _):
        ones_v[i] = jnp.ones((HID,), jnp.float32)
        return 0
    lax.fori_loop(0, CHUNK, ones_body, 0)

    pltpu.sync_copy(zeros_v, acc_sh.at[pl.ds(sid * STRIPE, STRIPE)])
    pltpu.sync_copy(edge_hbm.at[1, pl.ds(base, EPT)], dst_v)
    plsc.subcore_barrier()

    def fire(j, _):
        pltpu.async_copy(ones_v, acc_sh.at[dst_v.at[pl.ds(j * CHUNK, CHUNK)]],
                         sem, add=True)
        return 0
    lax.fori_loop(0, NFULL, fire, 0)
    pltpu.async_copy(ones_v.at[pl.ds(0, REM)],
                     acc_sh.at[dst_v.at[pl.ds(NFULL * CHUNK, REM)]], sem, add=True)

    def drain(j, _):
        pltpu.make_async_copy(ones_v, acc_sh.at[dst_v.at[pl.ds(0, CHUNK)]],
                              sem).wait()
        return 0
    lax.fori_loop(0, NFULL, drain, 0)
    pltpu.make_async_copy(ones_v.at[pl.ds(0, REM)],
                          acc_sh.at[dst_v.at[pl.ds(0, REM)]], sem).wait()

    plsc.subcore_barrier()
    pltpu.sync_copy(acc_sh.at[pl.ds(sid * STRIPE, STRIPE)],
                    out_hbm.at[cid, pl.ds(sid * STRIPE, STRIPE)])


@functools.partial(
    pl.kernel,
    out_type=jax.ShapeDtypeStruct((NC, PAD_NODES, HID), jnp.float32),
    mesh=_mesh,
    scratch_types=[
        pltpu.VMEM((EPT,), jnp.int32),
        pltpu.VMEM((EPT,), jnp.int32),
        pltpu.VMEM((2, NBUF, CHUNK, HID), jnp.float32),
        pltpu.VMEM((REM, HID), jnp.float32),
        pltpu.VMEM((STRIPE, HID), jnp.float32),
        pltpu.VMEM_SHARED((PAD_NODES, HID), jnp.float32),
        pltpu.VMEM_SHARED((PAD_NODES, HID), jnp.float32),
        pltpu.SemaphoreType.DMA,
        pltpu.SemaphoreType.DMA,
        pltpu.SemaphoreType.DMA,
        pltpu.SemaphoreType.DMA,
    ],
    compiler_params=pltpu.CompilerParams(use_tc_tiling_on_sc=False),
)
def _scatter_kernel(y_hbm, edge_hbm, out_hbm,
                    src_v, dst_v, rows_v, rem_v, zeros_v,
                    acc_sh, y_sh, gsem0, gsem1, ssem0, ssem1):
    cid = lax.axis_index("c")
    sid = lax.axis_index("s")
    wid = sid * NC + cid
    base = wid * EPT

    _zero_rows(zeros_v, STRIPE)
    pltpu.sync_copy(zeros_v, acc_sh.at[pl.ds(sid * STRIPE, STRIPE)])
    pltpu.sync_copy(y_hbm.at[pl.ds(sid * STRIPE, STRIPE)],
                    y_sh.at[pl.ds(sid * STRIPE, STRIPE)])
    pltpu.sync_copy(edge_hbm.at[0, pl.ds(base, EPT)], src_v)
    pltpu.sync_copy(edge_hbm.at[1, pl.ds(base, EPT)], dst_v)
    plsc.subcore_barrier()

    def sidx(j):
        return src_v.at[pl.ds(j * CHUNK, CHUNK)]

    def didx(j):
        return dst_v.at[pl.ds(j * CHUNK, CHUNK)]

    def fire_gathers(g, half, sem):
        base_c = g * NBUF
        for b in range(NBUF):
            pltpu.async_copy(y_sh.at[sidx(base_c + b)], rows_v.at[half, b], sem)

    def drain_gathers(g, half, sem):
        base_c = g * NBUF
        for b in range(NBUF):
            pltpu.make_async_copy(y_sh.at[sidx(base_c + b)],
                                  rows_v.at[half, b], sem).wait()

    def fire_scatters(g, half, sem):
        base_c = g * NBUF
        for b in range(NBUF):
            pltpu.async_copy(rows_v.at[half, b], acc_sh.at[didx(base_c + b)],
                             sem, add=True)

    def drain_scatters(g, half, sem):
        base_c = g * NBUF
        for b in range(NBUF):
            pltpu.make_async_copy(rows_v.at[half, b],
                                  acc_sh.at[didx(base_c + b)], sem).wait()

    fire_gathers(0, 0, gsem0)

    def body(gg, _):
        g0 = 2 * gg
        g1 = g0 + 1
        fire_gathers(g1, 1, gsem1)
        drain_gathers(g0, 0, gsem0)
        fire_scatters(g0, 0, ssem0)
        drain_scatters(g0, 0, ssem0)

        @pl.when(g0 + 2 < GROUPS)
        def _():
            fire_gathers(g0 + 2, 0, gsem0)

        drain_gathers(g1, 1, gsem1)
        fire_scatters(g1, 1, ssem1)
        drain_scatters(g1, 1, ssem1)
        return 0
    lax.fori_loop(0, GROUPS // 2, body, 0)

    rs = src_v.at[pl.ds(NFULL * CHUNK, REM)]
    rd = dst_v.at[pl.ds(NFULL * CHUNK, REM)]
    pltpu.async_copy(y_sh.at[rs], rem_v, gsem0).wait()
    pltpu.async_copy(rem_v, acc_sh.at[rd], ssem0, add=True).wait()

    plsc.subcore_barrier()
    pltpu.sync_copy(acc_sh.at[pl.ds(sid * STRIPE, STRIPE)],
                    out_hbm.at[cid, pl.ds(sid * STRIPE, STRIPE)])


NFLAT = PAD_NODES * HID // 128
NOUT = PAD_NODES // 128


def _tca_body(x_ref, w1_ref, xw_ref):
    xw = jnp.dot(x_ref[:], w1_ref[:], preferred_element_type=jnp.float32)
    xw_ref[:N_NODES] = xw
    xw_ref[N_NODES:] = jnp.zeros((PAD_NODES - N_NODES, HID), jnp.float32)


def _tcd_body(xwf_ref, degf_ref, yf_ref):
    dinvf = lax.rsqrt(degf_ref[0] + degf_ref[1] + 1.0)
    yf_ref[:] = dinvf * xwf_ref[:]


def _tcb_body(accf_ref, yf_ref, degf_ref, b1f_ref, w2bd_ref, b2_ref, out_ref):
    dinvf = lax.rsqrt(degf_ref[0] + degf_ref[1] + 1.0)
    s = accf_ref[0] + accf_ref[1] + yf_ref[:]
    h = jnp.maximum(dinvf * s + b1f_ref[:], 0.0)
    ovals = jnp.dot(h, w2bd_ref[:], preferred_element_type=jnp.float32)
    out_ref[:] = ovals + b2_ref[0, 0]


def kernel(x, edge_index, W1, b1, W2, b2):
    e32 = edge_index.astype(jnp.int32)

    degp = _deg_kernel(e32)
    degf = degp.reshape(NC, NFLAT, 128)

    xw = pl.pallas_call(
        _tca_body,
        out_shape=jax.ShapeDtypeStruct((PAD_NODES, HID), jnp.float32),
    )(x, W1)
    xwlin = xw.reshape(NFLAT, 128)

    ylin = pl.pallas_call(
        _tcd_body,
        out_shape=jax.ShapeDtypeStruct((NFLAT, 128), jnp.float32),
    )(xwlin, degf)

    acc = _scatter_kernel(ylin.reshape(PAD_NODES, HID), e32)
    accf = acc.reshape(NC, NFLAT, 128)

    w2bd = jnp.kron(jnp.eye(8, dtype=jnp.float32), W2)
    b1f = jnp.tile(b1, 8).reshape(1, 128)

    outp = pl.pallas_call(
        _tcb_body,
        out_shape=jax.ShapeDtypeStruct((NFLAT, 8), jnp.float32),
    )(accf, ylin, degf, b1f, w2bd, b2.reshape(1, 1))
    return outp.reshape(PAD_NODES)[:N_NODES]
